# baseline (speedup 1.0000x reference)
import functools

import jax
import jax.numpy as jnp
from jax import lax
from jax.experimental import pallas as pl
from jax.experimental.pallas import tpu as pltpu
from jax.experimental.pallas import tpu_sc as plsc

N = 10000
E = 320000
V = 512
D = 128

NC = 2
NS = 16
L = 16
NW = NC * NS

NP = 10240
K = 80
CE = E // K
CPW = CE // NW
CPT = CE // NS
DB = 25
IDR = NP // K
IDPW = IDR // NW
RPT = NP // NS

CB = K * D * 4
KB4 = K * 4
EB1 = 3 * KB4
EB2 = 3 * KB4 + KB4


def _zero16():
    return jnp.zeros((L,), jnp.float32)


def _inv_sqrt16(d):
    i = lax.bitcast_convert_type(d, jnp.int32)
    y = lax.bitcast_convert_type(jnp.int32(0x5F3759DF) - (i >> 1), jnp.float32)
    for _ in range(3):
        y = y * (1.5 - 0.5 * d * y * y)
    return y


def _zero_acc(acc_sp, xg_v, sub):
    def zrow(r, carry):
        for j in range(D // L):
            xg_v[0, r, pl.ds(j * L, L)] = _zero16()
        return carry
    lax.fori_loop(0, K, zrow, 0)
    for kblk in range(RPT // K):
        pltpu.sync_copy(xg_v.at[0],
                        acc_sp.at[pl.ds(sub * RPT + kblk * K, K), :])


def _writeout_acc(acc_sp, xg_v, agg_hbm, core, sub):
    for kblk in range(RPT // K):
        rb = sub * RPT + kblk * K
        pltpu.sync_copy(acc_sp.at[pl.ds(rb, K), :], xg_v.at[0])
        pltpu.sync_copy(xg_v.at[0], agg_hbm.at[core, pl.ds(rb, K), :])


def _make_scale(xg_v, nrm_v):
    def scale(slot):
        def ebody(e, carry):
            s16 = plsc.load_gather(
                nrm_v, [jnp.full((L,), slot, jnp.int32),
                        jnp.full((L,), e, jnp.int32)])
            for j in range(D // L):
                sl = pl.ds(j * L, L)
                xg_v[slot, e, sl] = xg_v[slot, e, sl] * s16
            return carry
        lax.fori_loop(0, K, ebody, 0, unroll=4)
    return scale


def _sc1_body(nidsf_hbm, epk_hbm, colx_hbm, ewx_hbm, emb_hbm,
              x_hbm, dis_hbm, norm_hbm, agg_hbm,
              deg_sp, dis_sp, acc_sp,
              ids_v, xg_v, ebuf, col_v, nrm_v, idx2_v, dcol, dew,
              nid_v, dis_v, dtmp,
              gsem0, gsem1, esem, ssem, nsem0, nsem1, dsem):
    core = lax.axis_index("c")
    sub = lax.axis_index("s")
    wid = core * NS + sub
    gsems = (gsem0, gsem1)
    nsems = (nsem0, nsem1)

    def z16(i, carry):
        dtmp[pl.ds(i * L, L)] = _zero16()
        return carry
    lax.fori_loop(0, RPT // L, z16, 0)
    pltpu.sync_copy(dtmp, deg_sp.at[pl.ds(sub * RPT, RPT)])
    _zero_acc(acc_sp, xg_v, sub)
    plsc.subcore_barrier()

    for kk in range(IDPW):
        r = wid * IDPW + kk
        pltpu.sync_copy(nidsf_hbm.at[pl.ds(r * K, K)], ids_v)
        pltpu.async_copy(emb_hbm.at[ids_v], xg_v.at[0], gsem0).wait()
        pltpu.sync_copy(xg_v.at[0], x_hbm.at[pl.ds(r * K, K), :])

    for b in range(CPT // DB):
        base = sub * CPT + b * DB
        pltpu.sync_copy(colx_hbm.at[pl.ds(base, DB)], dcol)
        pltpu.sync_copy(ewx_hbm.at[pl.ds(base, DB)], dew)

        def dbody(j, carry):
            pltpu.async_copy(dew.at[j], deg_sp.at[dcol.at[j]], dsem, add=True)
            return carry
        lax.fori_loop(0, DB, dbody, 0)

        def dwait(j, carry):
            pltpu.make_async_copy(dew.at[j], deg_sp.at[dcol.at[j]],
                                  dsem).wait()
            return carry
        lax.fori_loop(0, DB, dwait, 0)
    plsc.subcore_barrier()

    base = sub * RPT
    pltpu.sync_copy(deg_sp.at[pl.ds(base, RPT)], dtmp)

    def ibody(i, carry):
        sl = pl.ds(i * L, L)
        dtmp[sl] = _inv_sqrt16(dtmp[sl] + 1.0)
        return carry
    lax.fori_loop(0, RPT // L, ibody, 0)
    pltpu.sync_copy(dtmp, dis_sp.at[pl.ds(base, RPT)])

    @pl.when(core == 0)
    def _():
        pltpu.sync_copy(dtmp, dis_hbm.at[pl.ds(base, RPT)])
    plsc.subcore_barrier()

    pltpu.sync_copy(dis_sp, dis_v)
    pltpu.sync_copy(nidsf_hbm, nid_v)
    cb = wid * CPW
    scale = _make_scale(xg_v, nrm_v)

    def compute(slot):
        for j in range(K // L):
            sl = pl.ds(j * L, L)
            r16 = ebuf[slot, pl.ds(j * L, L)]
            c16 = ebuf[slot, pl.ds(K + j * L, L)]
            ew16 = lax.bitcast_convert_type(
                ebuf[slot, pl.ds(2 * K + j * L, L)], jnp.float32)
            dr = plsc.load_gather(dis_v, [r16])
            dc = plsc.load_gather(dis_v, [c16])
            nrm_v[slot, sl] = dr * ew16 * dc
            col_v[slot, sl] = c16
            idx2_v[slot, sl] = plsc.load_gather(nid_v, [r16])

    def wait_scatter(s):
        pltpu.make_async_copy(xg_v.at[s], acc_sp.at[col_v.at[s]],
                              ssem).wait()

    def wait_gather(s):
        pltpu.make_async_copy(emb_hbm.at[idx2_v.at[s]], xg_v.at[s],
                              gsems[s]).wait()

    def wait_norm(s, c1):
        pltpu.make_async_copy(nrm_v.at[s], norm_hbm.at[cb + c1],
                              nsems[s]).wait()

    def wait_ebuf(s, c1):
        pltpu.make_async_copy(epk_hbm.at[cb + c1], ebuf.at[s], esem).wait()

    def start_chunk(c1, q):
        compute(q)
        pltpu.async_copy(nrm_v.at[q], norm_hbm.at[cb + c1], nsems[q])
        pltpu.async_copy(emb_hbm.at[idx2_v.at[q]], xg_v.at[q], gsems[q])

    def finish_chunk(p):
        wait_gather(p)
        scale(p)
        pltpu.async_copy(xg_v.at[p], acc_sp.at[col_v.at[p]], ssem, add=True)

    def steady(c, p):
        q = 1 - p
        wait_scatter(q)
        wait_norm(q, c - 1)
        wait_ebuf(q, c + 1)
        start_chunk(c + 1, q)

        @pl.when(c + 2 < CPW)
        def _():
            pltpu.async_copy(epk_hbm.at[cb + c + 2], ebuf.at[p], esem)
        finish_chunk(p)

    pltpu.sync_copy(epk_hbm.at[cb], ebuf.at[0])
    start_chunk(0, 0)
    pltpu.async_copy(epk_hbm.at[cb + 1], ebuf.at[1], esem)
    wait_ebuf(1, 1)
    start_chunk(1, 1)
    pltpu.async_copy(epk_hbm.at[cb + 2], ebuf.at[0], esem)
    finish_chunk(0)
    steady(1, 1)

    def lbody(cc, carry):
        c = 2 + 2 * cc
        steady(c, 0)
        steady(c + 1, 1)
        return carry
    lax.fori_loop(0, (CPW - 3) // 2, lbody, 0)

    finish_chunk(0)
    wait_scatter(1)
    wait_scatter(0)
    wait_norm(1, CPW - 2)
    wait_norm(0, CPW - 1)
    plsc.subcore_barrier()

    _writeout_acc(acc_sp, xg_v, agg_hbm, core, sub)


def _sc2_body(y_hbm, epk_hbm, norm_hbm,
              agg_hbm,
              acc_sp,
              xg_v, ebuf, row_v, col_v, nrm_v,
              gsem0, gsem1, gsem2, esem, ssem):
    core = lax.axis_index("c")
    sub = lax.axis_index("s")
    wid = core * NS + sub
    gsems = (gsem0, gsem1, gsem2)

    _zero_acc(acc_sp, xg_v, sub)
    plsc.subcore_barrier()

    cb = wid * CPW
    scale = _make_scale(xg_v, nrm_v)

    def compute(slot):
        for j in range(K // L):
            sl = pl.ds(j * L, L)
            row_v[slot, sl] = ebuf[slot, pl.ds(j * L, L)]
            col_v[slot, sl] = ebuf[slot, pl.ds(K + j * L, L)]

    def wait_scatter(s):
        pltpu.make_async_copy(xg_v.at[s], acc_sp.at[col_v.at[s]],
                              ssem).wait()

    def wait_gather(s):
        pltpu.make_async_copy(y_hbm.at[row_v.at[s]], xg_v.at[s],
                              gsems[s]).wait()

    def wait_prefetch(s, c1):
        pltpu.make_async_copy(epk_hbm.at[cb + c1], ebuf.at[s], esem).wait()
        pltpu.make_async_copy(norm_hbm.at[cb + c1], nrm_v.at[s], esem).wait()

    def start_chunk(q):
        compute(q)
        pltpu.async_copy(y_hbm.at[row_v.at[q]], xg_v.at[q], gsems[q])

    def prefetch(c2, p):
        pltpu.async_copy(epk_hbm.at[cb + c2], ebuf.at[p], esem)
        pltpu.async_copy(norm_hbm.at[cb + c2], nrm_v.at[p], esem)

    def issue_scatter(s):
        pltpu.async_copy(xg_v.at[s], acc_sp.at[col_v.at[s]], ssem, add=True)

    def steady(c, s0, skip_scatter_wait=False):
        s1 = (s0 + 1) % 3
        sm = (s0 + 2) % 3
        wait_prefetch(s1, c + 1)
        if not skip_scatter_wait:
            wait_scatter(s1)
        start_chunk(s1)
        wait_gather(s0)
        scale(s0)

        @pl.when(c + 2 < CPW)
        def _():
            prefetch(c + 2, sm)
        issue_scatter(s0)

    pltpu.sync_copy(epk_hbm.at[cb], ebuf.at[0])
    pltpu.sync_copy(norm_hbm.at[cb], nrm_v.at[0])
    start_chunk(0)
    prefetch(1, 1)
    steady(0, 0, skip_scatter_wait=True)
    steady(1, 1, skip_scatter_wait=True)
    steady(2, 2)
    steady(3, 0)

    def lbody(cc, carry):
        c = 4 + 3 * cc
        steady(c, 1)
        steady(c + 1, 2)
        steady(c + 2, 0)
        return carry
    lax.fori_loop(0, (CPW - 5) // 3, lbody, 0)

    wait_gather(1)
    scale(1)
    issue_scatter(1)
    wait_scatter(2)
    wait_scatter(0)
    wait_scatter(1)
    plsc.subcore_barrier()

    _writeout_acc(acc_sp, xg_v, agg_hbm, core, sub)


_SC_MESH = plsc.VectorSubcoreMesh(core_axis_name="c", subcore_axis_name="s",
                                  num_cores=NC, num_subcores=NS)
_SC_PARAMS = pltpu.CompilerParams(needs_layout_passes=False,
                                  use_tc_tiling_on_sc=False)

_sc1 = pl.kernel(
    _sc1_body,
    out_type=(
        jax.ShapeDtypeStruct((NP, D), jnp.float32),
        jax.ShapeDtypeStruct((NP,), jnp.float32),
        jax.ShapeDtypeStruct((CE, K), jnp.float32),
        jax.ShapeDtypeStruct((NC, NP, D), jnp.float32),
    ),
    mesh=_SC_MESH,
    scratch_types=[
        pltpu.VMEM_SHARED((NP,), jnp.float32),
        pltpu.VMEM_SHARED((NP,), jnp.float32),
        pltpu.VMEM_SHARED((NP, D), jnp.float32),
        pltpu.VMEM((K,), jnp.int32),
        pltpu.VMEM((2, K, D), jnp.float32),
        pltpu.VMEM((2, 3 * K), jnp.int32),
        pltpu.VMEM((2, K), jnp.int32),
        pltpu.VMEM((2, K), jnp.float32),
        pltpu.VMEM((2, K), jnp.int32),
        pltpu.VMEM((DB, K), jnp.int32),
        pltpu.VMEM((DB, K), jnp.float32),
        pltpu.VMEM((NP,), jnp.int32),
        pltpu.VMEM((NP,), jnp.float32),
        pltpu.VMEM((RPT,), jnp.float32),
        pltpu.SemaphoreType.DMA,
        pltpu.SemaphoreType.DMA,
        pltpu.SemaphoreType.DMA,
        pltpu.SemaphoreType.DMA,
        pltpu.SemaphoreType.DMA,
        pltpu.SemaphoreType.DMA,
        pltpu.SemaphoreType.DMA,
    ],
    compiler_params=_SC_PARAMS,
)

_sc2 = pl.kernel(
    _sc2_body,
    out_type=(
        jax.ShapeDtypeStruct((NC, NP, D), jnp.float32),
    ),
    mesh=_SC_MESH,
    scratch_types=[
        pltpu.VMEM_SHARED((NP, D), jnp.float32),
        pltpu.VMEM((3, K, D), jnp.float32),
        pltpu.VMEM((3, 3 * K), jnp.int32),
        pltpu.VMEM((3, K), jnp.int32),
        pltpu.VMEM((3, K), jnp.int32),
        pltpu.VMEM((3, K), jnp.float32),
        pltpu.SemaphoreType.DMA,
        pltpu.SemaphoreType.DMA,
        pltpu.SemaphoreType.DMA,
        pltpu.SemaphoreType.DMA,
        pltpu.SemaphoreType.DMA,
    ],
    compiler_params=_SC_PARAMS,
)


def _tc_body(x_ref, p_ref, dis_ref, w_ref, b_ref, g_ref, be_ref, o_ref):
    xb = x_ref[...]
    agg = p_ref[0] + p_ref[1]
    d = dis_ref[...]
    pre = agg + (d * d) * xb
    h = lax.dot_general(pre, w_ref[...], (((1,), (1,)), ((), ())),
                        preferred_element_type=jnp.float32,
                        precision=lax.Precision.HIGHEST)
    t = xb + h + b_ref[...]
    m = jnp.mean(t, axis=1, keepdims=True)
    v = jnp.mean((t - m) * (t - m), axis=1, keepdims=True)
    o_ref[...] = (t - m) * lax.rsqrt(v + 1e-5) * g_ref[...] + be_ref[...]


_TC_R = 1280

_tc_layer = pl.pallas_call(
    _tc_body,
    out_shape=jax.ShapeDtypeStruct((NP, D), jnp.float32),
    grid=(NP // _TC_R,),
    in_specs=[
        pl.BlockSpec((_TC_R, D), lambda i: (i, 0)),
        pl.BlockSpec((NC, _TC_R, D), lambda i: (0, i, 0)),
        pl.BlockSpec((_TC_R, 1), lambda i: (i, 0)),
        pl.BlockSpec((D, D), lambda i: (0, 0)),
        pl.BlockSpec((1, D), lambda i: (0, 0)),
        pl.BlockSpec((1, D), lambda i: (0, 0)),
        pl.BlockSpec((1, D), lambda i: (0, 0)),
    ],
    out_specs=pl.BlockSpec((_TC_R, D), lambda i: (i, 0)),
)


def kernel(node_ids, edge_index, edge_weight, emb, W1, b1, W2, b2,
           ln1_g, ln1_b, ln2_g, ln2_b):
    nids = jnp.concatenate(
        [node_ids.astype(jnp.int32), jnp.zeros((NP - N,), jnp.int32)])
    row = edge_index[0].astype(jnp.int32).reshape(CE, K)
    col = edge_index[1].astype(jnp.int32).reshape(CE, K)
    ew = edge_weight.reshape(CE, K)
    ew_bits = lax.bitcast_convert_type(ew, jnp.int32)
    epk = jnp.concatenate([row, col, ew_bits], axis=1)

    x, dis, norm, p1 = _sc1(nids, epk, col, ew, emb)
    dis1 = dis.reshape(NP, 1)
    y1 = _tc_layer(x, p1, dis1, W1, b1.reshape(1, D), ln1_g.reshape(1, D),
                   ln1_b.reshape(1, D))
    (p2,) = _sc2(y1, epk, norm)
    out = _tc_layer(y1, p2, dis1, W2, b2.reshape(1, D), ln2_g.reshape(1, D),
                    ln2_b.reshape(1, D))
    return out[:N]


# speedup vs baseline: 1.1631x; 1.1631x over previous
import functools

import jax
import jax.numpy as jnp
from jax import lax
from jax.experimental import pallas as pl
from jax.experimental.pallas import tpu as pltpu
from jax.experimental.pallas import tpu_sc as plsc

N = 10000
E = 320000
V = 512
D = 128

NC = 2
NS = 16
L = 16
NW = NC * NS

NP = 10240
K = 80
CE = E // K
CPW = CE // NW
CPT = CE // NS
DB = 25
IDR = NP // K
IDPW = IDR // NW
RPT = NP // NS

CB = K * D * 4
KB4 = K * 4
EB1 = 3 * KB4
EB2 = 3 * KB4 + KB4


def _zero16():
    return jnp.zeros((L,), jnp.float32)


def _inv_sqrt16(d):
    i = lax.bitcast_convert_type(d, jnp.int32)
    y = lax.bitcast_convert_type(jnp.int32(0x5F3759DF) - (i >> 1), jnp.float32)
    for _ in range(3):
        y = y * (1.5 - 0.5 * d * y * y)
    return y


def _zero_acc(acc_sp, xg_v, sub):
    def zrow(r, carry):
        for j in range(D // L):
            xg_v[0, r, pl.ds(j * L, L)] = _zero16()
        return carry
    lax.fori_loop(0, K, zrow, 0)
    for kblk in range(RPT // K):
        pltpu.sync_copy(xg_v.at[0],
                        acc_sp.at[pl.ds(sub * RPT + kblk * K, K), :])


def _writeout_acc(acc_sp, xg_v, agg_hbm, core, sub):
    for kblk in range(RPT // K):
        rb = sub * RPT + kblk * K
        pltpu.sync_copy(acc_sp.at[pl.ds(rb, K), :], xg_v.at[0])
        pltpu.sync_copy(xg_v.at[0], agg_hbm.at[core, pl.ds(rb, K), :])


def _make_scale(xg_v, nrm_v):
    def scale(slot):
        def ebody(e, carry):
            s16 = plsc.load_gather(
                nrm_v, [jnp.full((L,), slot, jnp.int32),
                        jnp.full((L,), e, jnp.int32)])
            for j in range(D // L):
                sl = pl.ds(j * L, L)
                xg_v[slot, e, sl] = xg_v[slot, e, sl] * s16
            return carry
        lax.fori_loop(0, K, ebody, 0, unroll=4)
    return scale


QR = NP // 4
SFL = QR * V
DUMB = SFL
BT = 10
NB = CPT // BT
RPQ = QR // NS
WB = 8
WN = RPQ // WB


def _sc1_body(nidsf_hbm, epk_hbm, emb_hbm,
              x_hbm, dis_hbm, norm_hbm, s_hbm,
              deg_sp, dis_sp, s_sp,
              xg1, ebuf, nrmb, sidxb, nid_v, dis_v, dtmp, sbuf,
              gsem, esem, ssem, nsem):
    core = lax.axis_index("c")
    sub = lax.axis_index("s")
    wid = core * NS + sub

    def z16(i, carry):
        dtmp[pl.ds(i * L, L)] = _zero16()
        return carry
    lax.fori_loop(0, RPT // L, z16, 0)
    pltpu.sync_copy(dtmp, deg_sp.at[pl.ds(sub * RPT, RPT)])

    def zs(i, carry):
        sbuf[pl.ds(i * L, L)] = _zero16()
        return carry
    lax.fori_loop(0, (WB * V) // L, zs, 0)
    for blk in range(WN):
        pltpu.sync_copy(sbuf, s_sp.at[pl.ds((sub * RPQ + blk * WB) * V,
                                            WB * V)])
    plsc.subcore_barrier()

    for kk in range(IDPW):
        r = wid * IDPW + kk
        pltpu.sync_copy(nidsf_hbm.at[pl.ds(r * K, K)], sidxb.at[0, 0])
        pltpu.async_copy(emb_hbm.at[sidxb.at[0, 0]], xg1, gsem).wait()
        pltpu.sync_copy(xg1, x_hbm.at[pl.ds(r * K, K), :])

    pltpu.sync_copy(nidsf_hbm, nid_v)
    erow0 = sub * CPT

    def load_batch_sync(b):
        pltpu.sync_copy(epk_hbm.at[pl.ds(erow0 + b * BT, BT)], ebuf.at[b & 1])

    def prefetch_batch(b, s):
        pltpu.async_copy(epk_hbm.at[pl.ds(erow0 + b * BT, BT)], ebuf.at[s],
                         esem)

    def wait_batch(b, s):
        pltpu.make_async_copy(epk_hbm.at[pl.ds(erow0 + b * BT, BT)],
                              ebuf.at[s], esem).wait()

    def run_scan(chunk_fn, batch_fn, drain_fn):
        def iter_(b, p, drains):
            if drains:
                drain_fn(p, b - 2)

            def tbody(t, carry):
                chunk_fn(p, t)
                return carry
            lax.fori_loop(0, BT, tbody, 0)
            batch_fn(p, b)

            @pl.when(b + 2 < NB)
            def _():
                prefetch_batch(b + 2, p)

        load_batch_sync(0)
        prefetch_batch(1, 1)
        iter_(0, 0, False)
        wait_batch(1, 1)
        iter_(1, 1, False)
        wait_batch(2, 0)
        iter_(2, 0, True)

        def lbody(bb, carry):
            b = 3 + 2 * bb
            wait_batch(b, 1)
            iter_(b, 1, True)
            wait_batch(b + 1, 0)
            iter_(b + 1, 0, True)
            return carry
        lax.fori_loop(0, (NB - 3) // 2, lbody, 0)

        drain_fn(1, NB - 2)
        drain_fn(0, NB - 1)

    def deg_chunk(p, t):
        for j in range(K // L):
            sl = pl.ds(j * L, L)
            sidxb[p, t, sl] = ebuf[p, t, pl.ds(K + j * L, L)]
            nrmb[p, t, sl] = lax.bitcast_convert_type(
                ebuf[p, t, pl.ds(2 * K + j * L, L)], jnp.float32)
        pltpu.async_copy(nrmb.at[p, t], deg_sp.at[sidxb.at[p, t]], ssem,
                         add=True)

    def deg_drain(s, b):
        def tbody(t, carry):
            pltpu.make_async_copy(nrmb.at[s, t], deg_sp.at[sidxb.at[s, t]],
                                  ssem).wait()
            return carry
        lax.fori_loop(0, BT, tbody, 0)

    run_scan(deg_chunk, lambda p, b: None, deg_drain)
    plsc.subcore_barrier()

    base = sub * RPT
    pltpu.sync_copy(deg_sp.at[pl.ds(base, RPT)], dtmp)

    def ibody(i, carry):
        sl = pl.ds(i * L, L)
        dtmp[sl] = _inv_sqrt16(dtmp[sl] + 1.0)
        return carry
    lax.fori_loop(0, RPT // L, ibody, 0)
    pltpu.sync_copy(dtmp, dis_sp.at[pl.ds(base, RPT)])

    @pl.when(core == 0)
    def _():
        pltpu.sync_copy(dtmp, dis_hbm.at[pl.ds(base, RPT)])
    plsc.subcore_barrier()
    pltpu.sync_copy(dis_sp, dis_v)

    lanes = lax.iota(jnp.int32, L)
    for half in range(2):
        qoff = (core * 2 + half) * QR
        write_norms = half == 0

        def s_chunk(p, t):
            for j in range(K // L):
                sl = pl.ds(j * L, L)
                r16 = ebuf[p, t, pl.ds(j * L, L)]
                c16 = ebuf[p, t, pl.ds(K + j * L, L)]
                ew16 = lax.bitcast_convert_type(
                    ebuf[p, t, pl.ds(2 * K + j * L, L)], jnp.float32)
                dr = plsc.load_gather(dis_v, [r16])
                dc = plsc.load_gather(dis_v, [c16])
                nrmb[p, t, sl] = dr * ew16 * dc
                nidr = plsc.load_gather(nid_v, [r16])
                u = c16 - qoff
                valid = (u >= 0) & (u < QR)
                sidxb[p, t, sl] = jnp.where(valid, u * V + nidr,
                                            DUMB + j * L + lanes)
            pltpu.async_copy(nrmb.at[p, t], s_sp.at[sidxb.at[p, t]], ssem,
                             add=True)

        def s_batch(p, b):
            if write_norms:
                @pl.when(core == 0)
                def _():
                    pltpu.async_copy(
                        nrmb.at[p], norm_hbm.at[pl.ds(erow0 + b * BT, BT)],
                        nsem)

        def s_drain(s, b):
            def tbody(t, carry):
                pltpu.make_async_copy(nrmb.at[s, t], s_sp.at[sidxb.at[s, t]],
                                      ssem).wait()
                return carry
            lax.fori_loop(0, BT, tbody, 0)
            if write_norms:
                @pl.when(core == 0)
                def _():
                    pltpu.make_async_copy(
                        nrmb.at[s], norm_hbm.at[pl.ds(erow0 + b * BT, BT)],
                        nsem).wait()

        run_scan(s_chunk, s_batch, s_drain)
        plsc.subcore_barrier()

        qidx = core * 2 + half
        for blk in range(WN):
            off = (sub * RPQ + blk * WB) * V
            pltpu.sync_copy(s_sp.at[pl.ds(off, WB * V)], sbuf)
            pltpu.sync_copy(sbuf, s_hbm.at[qidx, pl.ds(off, WB * V)])
        if half == 0:
            def zs2(i, carry):
                sbuf[pl.ds(i * L, L)] = _zero16()
                return carry
            lax.fori_loop(0, (WB * V) // L, zs2, 0)
            for blk in range(WN):
                off = (sub * RPQ + blk * WB) * V
                pltpu.sync_copy(sbuf, s_sp.at[pl.ds(off, WB * V)])
            plsc.subcore_barrier()


def _sc2_body(y_hbm, epk_hbm, norm_hbm,
              agg_hbm,
              acc_sp,
              xg_v, ebuf, row_v, col_v, nrm_v,
              gsem0, gsem1, gsem2, esem, ssem):
    core = lax.axis_index("c")
    sub = lax.axis_index("s")
    wid = core * NS + sub
    gsems = (gsem0, gsem1, gsem2)

    _zero_acc(acc_sp, xg_v, sub)
    plsc.subcore_barrier()

    cb = wid * CPW
    scale = _make_scale(xg_v, nrm_v)

    def compute(slot):
        for j in range(K // L):
            sl = pl.ds(j * L, L)
            row_v[slot, sl] = ebuf[slot, pl.ds(j * L, L)]
            col_v[slot, sl] = ebuf[slot, pl.ds(K + j * L, L)]

    def wait_scatter(s):
        pltpu.make_async_copy(xg_v.at[s], acc_sp.at[col_v.at[s]],
                              ssem).wait()

    def wait_gather(s):
        pltpu.make_async_copy(y_hbm.at[row_v.at[s]], xg_v.at[s],
                              gsems[s]).wait()

    def wait_prefetch(s, c1):
        pltpu.make_async_copy(epk_hbm.at[cb + c1], ebuf.at[s], esem).wait()
        pltpu.make_async_copy(norm_hbm.at[cb + c1], nrm_v.at[s], esem).wait()

    def start_chunk(q):
        compute(q)
        pltpu.async_copy(y_hbm.at[row_v.at[q]], xg_v.at[q], gsems[q])

    def prefetch(c2, p):
        pltpu.async_copy(epk_hbm.at[cb + c2], ebuf.at[p], esem)
        pltpu.async_copy(norm_hbm.at[cb + c2], nrm_v.at[p], esem)

    def issue_scatter(s):
        pltpu.async_copy(xg_v.at[s], acc_sp.at[col_v.at[s]], ssem, add=True)

    def steady(c, s0, skip_scatter_wait=False):
        s1 = (s0 + 1) % 3
        sm = (s0 + 2) % 3
        wait_prefetch(s1, c + 1)
        if not skip_scatter_wait:
            wait_scatter(s1)
        start_chunk(s1)
        wait_gather(s0)
        scale(s0)

        @pl.when(c + 2 < CPW)
        def _():
            prefetch(c + 2, sm)
        issue_scatter(s0)

    pltpu.sync_copy(epk_hbm.at[cb], ebuf.at[0])
    pltpu.sync_copy(norm_hbm.at[cb], nrm_v.at[0])
    start_chunk(0)
    prefetch(1, 1)
    steady(0, 0, skip_scatter_wait=True)
    steady(1, 1, skip_scatter_wait=True)
    steady(2, 2)
    steady(3, 0)

    def lbody(cc, carry):
        c = 4 + 3 * cc
        steady(c, 1)
        steady(c + 1, 2)
        steady(c + 2, 0)
        return carry
    lax.fori_loop(0, (CPW - 5) // 3, lbody, 0)

    wait_gather(1)
    scale(1)
    issue_scatter(1)
    wait_scatter(2)
    wait_scatter(0)
    wait_scatter(1)
    plsc.subcore_barrier()

    _writeout_acc(acc_sp, xg_v, agg_hbm, core, sub)


_SC_MESH = plsc.VectorSubcoreMesh(core_axis_name="c", subcore_axis_name="s",
                                  num_cores=NC, num_subcores=NS)
_SC_PARAMS = pltpu.CompilerParams(needs_layout_passes=False,
                                  use_tc_tiling_on_sc=False)

_sc1 = pl.kernel(
    _sc1_body,
    out_type=(
        jax.ShapeDtypeStruct((NP, D), jnp.float32),
        jax.ShapeDtypeStruct((NP,), jnp.float32),
        jax.ShapeDtypeStruct((CE, K), jnp.float32),
        jax.ShapeDtypeStruct((4, SFL), jnp.float32),
    ),
    mesh=_SC_MESH,
    scratch_types=[
        pltpu.VMEM_SHARED((NP,), jnp.float32),
        pltpu.VMEM_SHARED((NP,), jnp.float32),
        pltpu.VMEM_SHARED((SFL + V,), jnp.float32),
        pltpu.VMEM((K, D), jnp.float32),
        pltpu.VMEM((2, BT, 3 * K), jnp.int32),
        pltpu.VMEM((2, BT, K), jnp.float32),
        pltpu.VMEM((2, BT, K), jnp.int32),
        pltpu.VMEM((NP,), jnp.int32),
        pltpu.VMEM((NP,), jnp.float32),
        pltpu.VMEM((RPT,), jnp.float32),
        pltpu.VMEM((WB * V,), jnp.float32),
        pltpu.SemaphoreType.DMA,
        pltpu.SemaphoreType.DMA,
        pltpu.SemaphoreType.DMA,
        pltpu.SemaphoreType.DMA,
    ],
    compiler_params=_SC_PARAMS,
)

_sc2 = pl.kernel(
    _sc2_body,
    out_type=(
        jax.ShapeDtypeStruct((NC, NP, D), jnp.float32),
    ),
    mesh=_SC_MESH,
    scratch_types=[
        pltpu.VMEM_SHARED((NP, D), jnp.float32),
        pltpu.VMEM((3, K, D), jnp.float32),
        pltpu.VMEM((3, 3 * K), jnp.int32),
        pltpu.VMEM((3, K), jnp.int32),
        pltpu.VMEM((3, K), jnp.int32),
        pltpu.VMEM((3, K), jnp.float32),
        pltpu.SemaphoreType.DMA,
        pltpu.SemaphoreType.DMA,
        pltpu.SemaphoreType.DMA,
        pltpu.SemaphoreType.DMA,
        pltpu.SemaphoreType.DMA,
    ],
    compiler_params=_SC_PARAMS,
)


def _tc1_body(x_ref, s_ref, e_ref, dis_ref, w_ref, b_ref, g_ref, be_ref,
              o_ref):
    xb = x_ref[...]
    agg = lax.dot_general(s_ref[...], e_ref[...], (((1,), (0,)), ((), ())),
                          preferred_element_type=jnp.float32,
                          precision=lax.Precision.HIGHEST)
    d = dis_ref[...]
    pre = agg + (d * d) * xb
    h = lax.dot_general(pre, w_ref[...], (((1,), (1,)), ((), ())),
                        preferred_element_type=jnp.float32,
                        precision=lax.Precision.HIGHEST)
    t = xb + h + b_ref[...]
    m = jnp.mean(t, axis=1, keepdims=True)
    v = jnp.mean((t - m) * (t - m), axis=1, keepdims=True)
    o_ref[...] = (t - m) * lax.rsqrt(v + 1e-5) * g_ref[...] + be_ref[...]


def _tc_body(x_ref, p_ref, dis_ref, w_ref, b_ref, g_ref, be_ref, o_ref):
    xb = x_ref[...]
    agg = p_ref[0] + p_ref[1]
    d = dis_ref[...]
    pre = agg + (d * d) * xb
    h = lax.dot_general(pre, w_ref[...], (((1,), (1,)), ((), ())),
                        preferred_element_type=jnp.float32,
                        precision=lax.Precision.HIGHEST)
    t = xb + h + b_ref[...]
    m = jnp.mean(t, axis=1, keepdims=True)
    v = jnp.mean((t - m) * (t - m), axis=1, keepdims=True)
    o_ref[...] = (t - m) * lax.rsqrt(v + 1e-5) * g_ref[...] + be_ref[...]


_TC_R = 1280

_tc_layer1 = pl.pallas_call(
    _tc1_body,
    out_shape=jax.ShapeDtypeStruct((NP, D), jnp.float32),
    grid=(NP // _TC_R,),
    in_specs=[
        pl.BlockSpec((_TC_R, D), lambda i: (i, 0)),
        pl.BlockSpec((_TC_R, V), lambda i: (i, 0)),
        pl.BlockSpec((V, D), lambda i: (0, 0)),
        pl.BlockSpec((_TC_R, 1), lambda i: (i, 0)),
        pl.BlockSpec((D, D), lambda i: (0, 0)),
        pl.BlockSpec((1, D), lambda i: (0, 0)),
        pl.BlockSpec((1, D), lambda i: (0, 0)),
        pl.BlockSpec((1, D), lambda i: (0, 0)),
    ],
    out_specs=pl.BlockSpec((_TC_R, D), lambda i: (i, 0)),
)

_tc_layer = pl.pallas_call(
    _tc_body,
    out_shape=jax.ShapeDtypeStruct((NP, D), jnp.float32),
    grid=(NP // _TC_R,),
    in_specs=[
        pl.BlockSpec((_TC_R, D), lambda i: (i, 0)),
        pl.BlockSpec((NC, _TC_R, D), lambda i: (0, i, 0)),
        pl.BlockSpec((_TC_R, 1), lambda i: (i, 0)),
        pl.BlockSpec((D, D), lambda i: (0, 0)),
        pl.BlockSpec((1, D), lambda i: (0, 0)),
        pl.BlockSpec((1, D), lambda i: (0, 0)),
        pl.BlockSpec((1, D), lambda i: (0, 0)),
    ],
    out_specs=pl.BlockSpec((_TC_R, D), lambda i: (i, 0)),
)


def kernel(node_ids, edge_index, edge_weight, emb, W1, b1, W2, b2,
           ln1_g, ln1_b, ln2_g, ln2_b):
    nids = jnp.concatenate(
        [node_ids.astype(jnp.int32), jnp.zeros((NP - N,), jnp.int32)])
    row = edge_index[0].astype(jnp.int32).reshape(CE, K)
    col = edge_index[1].astype(jnp.int32).reshape(CE, K)
    ew = edge_weight.reshape(CE, K)
    ew_bits = lax.bitcast_convert_type(ew, jnp.int32)
    epk = jnp.concatenate([row, col, ew_bits], axis=1)

    x, dis, norm, s_q = _sc1(nids, epk, emb)
    s_full = s_q.reshape(NP, V)
    dis1 = dis.reshape(NP, 1)
    y1 = _tc_layer1(x, s_full, emb, dis1, W1, b1.reshape(1, D),
                    ln1_g.reshape(1, D), ln1_b.reshape(1, D))
    (p2,) = _sc2(y1, epk, norm)
    out = _tc_layer(y1, p2, dis1, W2, b2.reshape(1, D), ln2_g.reshape(1, D),
                    ln2_b.reshape(1, D))
    return out[:N]


# speedup vs baseline: 1.2083x; 1.0389x over previous
import functools

import jax
import jax.numpy as jnp
from jax import lax
from jax.experimental import pallas as pl
from jax.experimental.pallas import tpu as pltpu
from jax.experimental.pallas import tpu_sc as plsc

N = 10000
E = 320000
V = 512
D = 128

NC = 2
NS = 16
L = 16
NW = NC * NS

NP = 10240
K = 80
CE = E // K
CPW = CE // NW
CPT = CE // NS
DB = 25
IDR = NP // K
IDPW = IDR // NW
RPT = NP // NS

CB = K * D * 4
KB4 = K * 4
EB1 = 3 * KB4
EB2 = 3 * KB4 + KB4


def _zero16():
    return jnp.zeros((L,), jnp.float32)


def _inv_sqrt16(d):
    i = lax.bitcast_convert_type(d, jnp.int32)
    y = lax.bitcast_convert_type(jnp.int32(0x5F3759DF) - (i >> 1), jnp.float32)
    for _ in range(3):
        y = y * (1.5 - 0.5 * d * y * y)
    return y


def _zero_acc(acc_sp, xg_v, sub):
    def zrow(r, carry):
        for j in range(D // L):
            xg_v[0, r, pl.ds(j * L, L)] = _zero16()
        return carry
    lax.fori_loop(0, K, zrow, 0)
    for kblk in range(RPT // K):
        pltpu.sync_copy(xg_v.at[0],
                        acc_sp.at[pl.ds(sub * RPT + kblk * K, K), :])


def _writeout_acc(acc_sp, xg_v, agg_hbm, core, sub):
    for kblk in range(RPT // K):
        rb = sub * RPT + kblk * K
        pltpu.sync_copy(acc_sp.at[pl.ds(rb, K), :], xg_v.at[0])
        pltpu.sync_copy(xg_v.at[0], agg_hbm.at[core, pl.ds(rb, K), :])


def _make_scale(xg_v, nrm_v):
    def scale(slot):
        def ebody(e, carry):
            s16 = plsc.load_gather(
                nrm_v, [jnp.full((L,), slot, jnp.int32),
                        jnp.full((L,), e, jnp.int32)])
            for j in range(D // L):
                sl = pl.ds(j * L, L)
                xg_v[slot, e, sl] = xg_v[slot, e, sl] * s16
            return carry
        lax.fori_loop(0, K, ebody, 0, unroll=4)
    return scale


QR = NP // 4
SFL = QR * V
DUMB = SFL
BT = 10
NB = CPT // BT
RPQ = QR // NS
WB = 8
WN = RPQ // WB


def _sc1_body(nidsf_hbm, epk_hbm, emb_hbm,
              x_hbm, dis_hbm, norm_hbm, s_hbm,
              deg_sp, dis_sp, s_sp,
              xg1, ebuf, nrmb, sidxb, nid_v, dis_v, dtmp, sbuf,
              gsem, esem, ssem, nsem):
    core = lax.axis_index("c")
    sub = lax.axis_index("s")
    wid = core * NS + sub

    def z16(i, carry):
        dtmp[pl.ds(i * L, L)] = _zero16()
        return carry
    lax.fori_loop(0, RPT // L, z16, 0)
    pltpu.sync_copy(dtmp, deg_sp.at[pl.ds(sub * RPT, RPT)])

    def zs(i, carry):
        sbuf[pl.ds(i * L, L)] = _zero16()
        return carry
    lax.fori_loop(0, (WB * V) // L, zs, 0)
    for blk in range(WN):
        pltpu.sync_copy(sbuf, s_sp.at[pl.ds((sub * RPQ + blk * WB) * V,
                                            WB * V)])
    plsc.subcore_barrier()

    for kk in range(IDPW):
        r = wid * IDPW + kk
        pltpu.sync_copy(nidsf_hbm.at[pl.ds(r * K, K)], sidxb.at[0, 0])
        pltpu.async_copy(emb_hbm.at[sidxb.at[0, 0]], xg1, gsem).wait()
        pltpu.sync_copy(xg1, x_hbm.at[pl.ds(r * K, K), :])

    pltpu.sync_copy(nidsf_hbm, nid_v)
    erow0 = sub * CPT

    def load_batch_sync(b):
        pltpu.sync_copy(epk_hbm.at[pl.ds(erow0 + b * BT, BT)], ebuf.at[b & 1])

    def prefetch_batch(b, s):
        pltpu.async_copy(epk_hbm.at[pl.ds(erow0 + b * BT, BT)], ebuf.at[s],
                         esem)

    def wait_batch(b, s):
        pltpu.make_async_copy(epk_hbm.at[pl.ds(erow0 + b * BT, BT)],
                              ebuf.at[s], esem).wait()

    def run_scan(chunk_fn, batch_fn, drain_fn):
        def iter_(b, p, drains):
            if drains:
                drain_fn(p, b - 2)

            def tbody(t, carry):
                chunk_fn(p, t)
                return carry
            lax.fori_loop(0, BT, tbody, 0)
            batch_fn(p, b)

            @pl.when(b + 2 < NB)
            def _():
                prefetch_batch(b + 2, p)

        load_batch_sync(0)
        prefetch_batch(1, 1)
        iter_(0, 0, False)
        wait_batch(1, 1)
        iter_(1, 1, False)
        wait_batch(2, 0)
        iter_(2, 0, True)

        def lbody(bb, carry):
            b = 3 + 2 * bb
            wait_batch(b, 1)
            iter_(b, 1, True)
            wait_batch(b + 1, 0)
            iter_(b + 1, 0, True)
            return carry
        lax.fori_loop(0, (NB - 3) // 2, lbody, 0)

        drain_fn(1, NB - 2)
        drain_fn(0, NB - 1)

    def deg_chunk(p, t):
        for j in range(K // L):
            sl = pl.ds(j * L, L)
            sidxb[p, t, sl] = ebuf[p, t, pl.ds(K + j * L, L)]
            nrmb[p, t, sl] = lax.bitcast_convert_type(
                ebuf[p, t, pl.ds(2 * K + j * L, L)], jnp.float32)
        pltpu.async_copy(nrmb.at[p, t], deg_sp.at[sidxb.at[p, t]], ssem,
                         add=True)

    def deg_drain(s, b):
        def tbody(t, carry):
            pltpu.make_async_copy(nrmb.at[s, t], deg_sp.at[sidxb.at[s, t]],
                                  ssem).wait()
            return carry
        lax.fori_loop(0, BT, tbody, 0)

    run_scan(deg_chunk, lambda p, b: None, deg_drain)
    plsc.subcore_barrier()

    base = sub * RPT
    pltpu.sync_copy(deg_sp.at[pl.ds(base, RPT)], dtmp)

    def ibody(i, carry):
        sl = pl.ds(i * L, L)
        dtmp[sl] = _inv_sqrt16(dtmp[sl] + 1.0)
        return carry
    lax.fori_loop(0, RPT // L, ibody, 0)
    pltpu.sync_copy(dtmp, dis_sp.at[pl.ds(base, RPT)])

    @pl.when(core == 0)
    def _():
        pltpu.sync_copy(dtmp, dis_hbm.at[pl.ds(base, RPT)])
    plsc.subcore_barrier()
    pltpu.sync_copy(dis_sp, dis_v)

    lanes = lax.iota(jnp.int32, L)
    for half in range(2):
        qoff = (core * 2 + half) * QR
        write_norms = half == 0

        def s_chunk(p, t):
            for j in range(K // L):
                sl = pl.ds(j * L, L)
                r16 = ebuf[p, t, pl.ds(j * L, L)]
                c16 = ebuf[p, t, pl.ds(K + j * L, L)]
                ew16 = lax.bitcast_convert_type(
                    ebuf[p, t, pl.ds(2 * K + j * L, L)], jnp.float32)
                dr = plsc.load_gather(dis_v, [r16])
                dc = plsc.load_gather(dis_v, [c16])
                nrmb[p, t, sl] = dr * ew16 * dc
                nidr = plsc.load_gather(nid_v, [r16])
                u = c16 - qoff
                valid = (u >= 0) & (u < QR)
                sidxb[p, t, sl] = jnp.where(valid, u * V + nidr,
                                            DUMB + j * L + lanes)
            pltpu.async_copy(nrmb.at[p, t], s_sp.at[sidxb.at[p, t]], ssem,
                             add=True)

        def s_batch(p, b):
            if write_norms:
                @pl.when(core == 0)
                def _():
                    pltpu.async_copy(
                        nrmb.at[p], norm_hbm.at[pl.ds(erow0 + b * BT, BT)],
                        nsem)

        def s_drain(s, b):
            def tbody(t, carry):
                pltpu.make_async_copy(nrmb.at[s, t], s_sp.at[sidxb.at[s, t]],
                                      ssem).wait()
                return carry
            lax.fori_loop(0, BT, tbody, 0)
            if write_norms:
                @pl.when(core == 0)
                def _():
                    pltpu.make_async_copy(
                        nrmb.at[s], norm_hbm.at[pl.ds(erow0 + b * BT, BT)],
                        nsem).wait()

        run_scan(s_chunk, s_batch, s_drain)
        plsc.subcore_barrier()

        qidx = core * 2 + half
        for blk in range(WN):
            off = (sub * RPQ + blk * WB) * V
            pltpu.sync_copy(s_sp.at[pl.ds(off, WB * V)], sbuf)
            pltpu.sync_copy(sbuf, s_hbm.at[qidx, pl.ds(off, WB * V)])
        if half == 0:
            def zs2(i, carry):
                sbuf[pl.ds(i * L, L)] = _zero16()
                return carry
            lax.fori_loop(0, (WB * V) // L, zs2, 0)
            for blk in range(WN):
                off = (sub * RPQ + blk * WB) * V
                pltpu.sync_copy(sbuf, s_sp.at[pl.ds(off, WB * V)])
            plsc.subcore_barrier()


def _sc2_body(y_hbm, epk_hbm, norm_hbm,
              agg_hbm,
              acc_sp,
              xg_v, ebuf, row_v, col_v, nrm_v,
              gsem0, gsem1, gsem2, esem, ssem):
    core = lax.axis_index("c")
    sub = lax.axis_index("s")
    wid = core * NS + sub
    gsems = (gsem0, gsem1, gsem2)

    _zero_acc(acc_sp, xg_v, sub)
    plsc.subcore_barrier()

    cb = wid * CPW
    scale = _make_scale(xg_v, nrm_v)

    def compute(slot):
        for j in range(K // L):
            sl = pl.ds(j * L, L)
            row_v[slot, sl] = ebuf[slot, pl.ds(j * L, L)]
            col_v[slot, sl] = ebuf[slot, pl.ds(K + j * L, L)]

    def wait_scatter(s):
        pltpu.make_async_copy(xg_v.at[s], acc_sp.at[col_v.at[s]],
                              ssem).wait()

    def wait_gather(s):
        pltpu.make_async_copy(y_hbm.at[row_v.at[s]], xg_v.at[s],
                              gsems[s]).wait()

    def wait_prefetch(s, c1):
        pltpu.make_async_copy(epk_hbm.at[cb + c1], ebuf.at[s], esem).wait()
        pltpu.make_async_copy(norm_hbm.at[cb + c1], nrm_v.at[s], esem).wait()

    def start_chunk(q):
        compute(q)
        pltpu.async_copy(y_hbm.at[row_v.at[q]], xg_v.at[q], gsems[q])

    def prefetch(c2, p):
        pltpu.async_copy(epk_hbm.at[cb + c2], ebuf.at[p], esem)
        pltpu.async_copy(norm_hbm.at[cb + c2], nrm_v.at[p], esem)

    def issue_scatter(s):
        pltpu.async_copy(xg_v.at[s], acc_sp.at[col_v.at[s]], ssem, add=True)

    def steady(c, s0, skip_scatter_wait=False):
        s1 = (s0 + 1) % 3
        sm = (s0 + 2) % 3
        wait_prefetch(s1, c + 1)
        if not skip_scatter_wait:
            wait_scatter(s1)
        start_chunk(s1)
        wait_gather(s0)
        scale(s0)

        @pl.when(c + 2 < CPW)
        def _():
            prefetch(c + 2, sm)
        issue_scatter(s0)

    pltpu.sync_copy(epk_hbm.at[cb], ebuf.at[0])
    pltpu.sync_copy(norm_hbm.at[cb], nrm_v.at[0])
    start_chunk(0)
    prefetch(1, 1)
    steady(0, 0, skip_scatter_wait=True)
    steady(1, 1, skip_scatter_wait=True)
    steady(2, 2)
    steady(3, 0)

    def lbody(cc, carry):
        c = 4 + 3 * cc
        steady(c, 1)
        steady(c + 1, 2)
        steady(c + 2, 0)
        return carry
    lax.fori_loop(0, (CPW - 5) // 3, lbody, 0)

    wait_gather(1)
    scale(1)
    issue_scatter(1)
    wait_scatter(2)
    wait_scatter(0)
    wait_scatter(1)
    plsc.subcore_barrier()

    _writeout_acc(acc_sp, xg_v, agg_hbm, core, sub)


_SC_MESH = plsc.VectorSubcoreMesh(core_axis_name="c", subcore_axis_name="s",
                                  num_cores=NC, num_subcores=NS)
_SC_PARAMS = pltpu.CompilerParams(needs_layout_passes=False,
                                  use_tc_tiling_on_sc=False)

_sc1 = pl.kernel(
    _sc1_body,
    out_type=(
        jax.ShapeDtypeStruct((NP, D), jnp.float32),
        jax.ShapeDtypeStruct((NP,), jnp.float32),
        jax.ShapeDtypeStruct((CE, K), jnp.float32),
        jax.ShapeDtypeStruct((4, SFL), jnp.float32),
    ),
    mesh=_SC_MESH,
    scratch_types=[
        pltpu.VMEM_SHARED((NP,), jnp.float32),
        pltpu.VMEM_SHARED((NP,), jnp.float32),
        pltpu.VMEM_SHARED((SFL + V,), jnp.float32),
        pltpu.VMEM((K, D), jnp.float32),
        pltpu.VMEM((2, BT, 3 * K), jnp.int32),
        pltpu.VMEM((2, BT, K), jnp.float32),
        pltpu.VMEM((2, BT, K), jnp.int32),
        pltpu.VMEM((NP,), jnp.int32),
        pltpu.VMEM((NP,), jnp.float32),
        pltpu.VMEM((RPT,), jnp.float32),
        pltpu.VMEM((WB * V,), jnp.float32),
        pltpu.SemaphoreType.DMA,
        pltpu.SemaphoreType.DMA,
        pltpu.SemaphoreType.DMA,
        pltpu.SemaphoreType.DMA,
    ],
    compiler_params=_SC_PARAMS,
)

_sc2 = pl.kernel(
    _sc2_body,
    out_type=(
        jax.ShapeDtypeStruct((NC, NP, D), jnp.float32),
    ),
    mesh=_SC_MESH,
    scratch_types=[
        pltpu.VMEM_SHARED((NP, D), jnp.float32),
        pltpu.VMEM((3, K, D), jnp.float32),
        pltpu.VMEM((3, 3 * K), jnp.int32),
        pltpu.VMEM((3, K), jnp.int32),
        pltpu.VMEM((3, K), jnp.int32),
        pltpu.VMEM((3, K), jnp.float32),
        pltpu.SemaphoreType.DMA,
        pltpu.SemaphoreType.DMA,
        pltpu.SemaphoreType.DMA,
        pltpu.SemaphoreType.DMA,
        pltpu.SemaphoreType.DMA,
    ],
    compiler_params=_SC_PARAMS,
)


def _tc1_body(x_ref, s_ref, e_ref, dis_ref, w_ref, b_ref, g_ref, be_ref,
              o_ref):
    xb = x_ref[...]
    agg = lax.dot_general(s_ref[...], e_ref[...], (((1,), (0,)), ((), ())),
                          preferred_element_type=jnp.float32)
    d = dis_ref[...]
    pre = agg + (d * d) * xb
    h = lax.dot_general(pre, w_ref[...], (((1,), (1,)), ((), ())),
                        preferred_element_type=jnp.float32,
                        precision=lax.Precision.HIGHEST)
    t = xb + h + b_ref[...]
    m = jnp.mean(t, axis=1, keepdims=True)
    v = jnp.mean((t - m) * (t - m), axis=1, keepdims=True)
    o_ref[...] = (t - m) * lax.rsqrt(v + 1e-5) * g_ref[...] + be_ref[...]


def _tc_body(x_ref, p_ref, dis_ref, w_ref, b_ref, g_ref, be_ref, o_ref):
    xb = x_ref[...]
    agg = p_ref[0] + p_ref[1]
    d = dis_ref[...]
    pre = agg + (d * d) * xb
    h = lax.dot_general(pre, w_ref[...], (((1,), (1,)), ((), ())),
                        preferred_element_type=jnp.float32,
                        precision=lax.Precision.HIGHEST)
    t = xb + h + b_ref[...]
    m = jnp.mean(t, axis=1, keepdims=True)
    v = jnp.mean((t - m) * (t - m), axis=1, keepdims=True)
    o_ref[...] = (t - m) * lax.rsqrt(v + 1e-5) * g_ref[...] + be_ref[...]


_TC_R = 1280

_tc_layer1 = pl.pallas_call(
    _tc1_body,
    out_shape=jax.ShapeDtypeStruct((NP, D), jnp.float32),
    grid=(NP // _TC_R,),
    in_specs=[
        pl.BlockSpec((_TC_R, D), lambda i: (i, 0)),
        pl.BlockSpec((_TC_R, V), lambda i: (i, 0)),
        pl.BlockSpec((V, D), lambda i: (0, 0)),
        pl.BlockSpec((_TC_R, 1), lambda i: (i, 0)),
        pl.BlockSpec((D, D), lambda i: (0, 0)),
        pl.BlockSpec((1, D), lambda i: (0, 0)),
        pl.BlockSpec((1, D), lambda i: (0, 0)),
        pl.BlockSpec((1, D), lambda i: (0, 0)),
    ],
    out_specs=pl.BlockSpec((_TC_R, D), lambda i: (i, 0)),
)

_tc_layer = pl.pallas_call(
    _tc_body,
    out_shape=jax.ShapeDtypeStruct((NP, D), jnp.float32),
    grid=(NP // _TC_R,),
    in_specs=[
        pl.BlockSpec((_TC_R, D), lambda i: (i, 0)),
        pl.BlockSpec((NC, _TC_R, D), lambda i: (0, i, 0)),
        pl.BlockSpec((_TC_R, 1), lambda i: (i, 0)),
        pl.BlockSpec((D, D), lambda i: (0, 0)),
        pl.BlockSpec((1, D), lambda i: (0, 0)),
        pl.BlockSpec((1, D), lambda i: (0, 0)),
        pl.BlockSpec((1, D), lambda i: (0, 0)),
    ],
    out_specs=pl.BlockSpec((_TC_R, D), lambda i: (i, 0)),
)


def kernel(node_ids, edge_index, edge_weight, emb, W1, b1, W2, b2,
           ln1_g, ln1_b, ln2_g, ln2_b):
    nids = jnp.concatenate(
        [node_ids.astype(jnp.int32), jnp.zeros((NP - N,), jnp.int32)])
    row = edge_index[0].astype(jnp.int32).reshape(CE, K)
    col = edge_index[1].astype(jnp.int32).reshape(CE, K)
    ew = edge_weight.reshape(CE, K)
    ew_bits = lax.bitcast_convert_type(ew, jnp.int32)
    epk = jnp.concatenate([row, col, ew_bits], axis=1)

    x, dis, norm, s_q = _sc1(nids, epk, emb)
    s_full = s_q.reshape(NP, V)
    dis1 = dis.reshape(NP, 1)
    y1 = _tc_layer1(x, s_full, emb, dis1, W1, b1.reshape(1, D),
                    ln1_g.reshape(1, D), ln1_b.reshape(1, D))
    (p2,) = _sc2(y1, epk, norm)
    out = _tc_layer(y1, p2, dis1, W2, b2.reshape(1, D), ln2_g.reshape(1, D),
                    ln2_b.reshape(1, D))
    return out[:N]
